# baseline (speedup 1.0000x reference)
import jax
import jax.numpy as jnp
from jax import lax
from jax.experimental import pallas as pl
from jax.experimental.pallas import tpu as pltpu
from jax.experimental.pallas import tpu_sc as plsc

_NC = 2
_NS = 16
_NW = _NC * _NS
_LANES = 16

_F = 26
_V = 100000
_D = 32
_B = 16384
_CH = 4096


def _body(xT_hbm, t3_hbm, outT_hbm, row_v, idx_v, out_v, sem_q0,
          sem_q1, sem_q2, sem_q3, sem_o0, sem_o1):
    sem_q = [sem_q0, sem_q1, sem_q2, sem_q3]
    w = lax.axis_index("s") * _NC + lax.axis_index("c")
    sem_o = [sem_o0, sem_o1]
    n_ch = _B // _CH

    zero16 = jnp.zeros((_LANES,), jnp.int32)

    def pair(i, prev_j):
        jd = _F * w + i
        j = jd // _D

        @pl.when(j != prev_j)
        def _():
            pltpu.sync_copy(xT_hbm.at[j], idx_v)
        qb = [0, 25088, 50176, 75264, _V]
        hq = [
            pltpu.async_copy(
                t3_hbm.at[pl.ds(jd, 1), pl.ds(qb[q], qb[q + 1] - qb[q])],
                row_v.at[:, pl.ds(qb[q], qb[q + 1] - qb[q])],
                sem_q[q],
            )
            for q in range(4)
        ]
        for h in hq:
            h.wait()
        stores = [None, None]
        for c in range(n_ch):
            @plsc.parallel_loop(0, _CH // _LANES, unroll=16)
            def gath(g):
                iv = idx_v[pl.ds(c * _CH + g * _LANES, _LANES)]
                out_v[c % 2, pl.ds(g * _LANES, _LANES)] = (
                    plsc.load_gather(row_v, [zero16, iv]))

            if stores[c % 2] is not None:
                stores[c % 2].wait()
            stores[c % 2] = pltpu.async_copy(
                out_v.at[c % 2],
                outT_hbm.at[jd, pl.ds(c * _CH, _CH)],
                sem_o[c % 2],
            )
        stores[(n_ch - 2) % 2].wait()
        stores[(n_ch - 1) % 2].wait()
        return j

    lax.fori_loop(0, _F, pair, jnp.int32(-1))


def kernel(x_cat, tables):
    B, F = x_cat.shape
    _, V, D = tables.shape
    xT = x_cat.T
    t3 = tables.transpose(0, 2, 1).reshape(F * D, V)

    k = pl.kernel(
        _body,
        out_type=jax.ShapeDtypeStruct((F * D, B), jnp.float32),
        mesh=plsc.VectorSubcoreMesh(core_axis_name="c", subcore_axis_name="s"),
        scratch_types=[
            pltpu.VMEM((1, V), jnp.float32),
            pltpu.VMEM((B,), jnp.int32),
            pltpu.VMEM((2, _CH), jnp.float32),
        ] + [pltpu.SemaphoreType.DMA] * 6,
        compiler_params=pltpu.CompilerParams(
            use_tc_tiling_on_sc=True, needs_layout_passes=False
        ),
    )
    outT = k(xT, t3)
    return outT.T
